# Initial kernel scaffold; baseline (speedup 1.0000x reference)
import jax
import jax.numpy as jnp
from jax.experimental import pallas as pl


def kernel(x, w1, b1, w2, b2):
    raise NotImplementedError("write your pallas kernel here")


# speedup vs baseline: 1.1521x; 1.1521x over previous
import functools

import jax
import jax.numpy as jnp
from jax.experimental import pallas as pl
from jax.experimental.pallas import tpu as pltpu


def _se_body(x_ref, w1_ref, b1_ref, w2t_ref, b2_ref, o_ref, *, mean_scale):
    xs = x_ref[...]

    col = jnp.sum(xs, axis=2, keepdims=True, dtype=jnp.float32) * mean_scale

    prod1 = col * w1_ref[...][None]
    hid = jnp.sum(prod1, axis=1, keepdims=True) + b1_ref[...]
    hid = jnp.maximum(hid, 0.0)

    prod2 = hid * w2t_ref[...][None]
    act = jnp.sum(prod2, axis=2, keepdims=True) + b2_ref[...][None]

    gate = jax.nn.sigmoid(act)

    o_ref[...] = xs * gate.astype(o_ref.dtype)


def _pick_tile(B, per_image_bytes, target_bytes=4 * 1024 * 1024):
    tile = 1
    for cand in range(2, B + 1):
        if B % cand:
            continue
        if cand * per_image_bytes > target_bytes or B // cand < 4:
            break
        tile = cand
    return tile


@jax.jit
def kernel(x, w1, b1, w2, b2):
    B, C, H, W = x.shape
    Cr = w1.shape[1]
    S = H * W

    xs = x.reshape(B, C, S)
    tile = _pick_tile(B, C * S * x.dtype.itemsize)

    block_bytes = tile * C * S * x.dtype.itemsize
    vmem_limit = min(6 * block_bytes + 8 * 1024 * 1024, 100 * 1024 * 1024)

    body = functools.partial(_se_body, mean_scale=1.0 / S)
    out = pl.pallas_call(
        body,
        out_shape=jax.ShapeDtypeStruct((B, C, S), x.dtype),
        grid=(B // tile,),
        in_specs=[
            pl.BlockSpec((tile, C, S), lambda b: (b, 0, 0)),
            pl.BlockSpec((C, Cr), lambda b: (0, 0)),
            pl.BlockSpec((1, 1, Cr), lambda b: (0, 0, 0)),
            pl.BlockSpec((C, Cr), lambda b: (0, 0)),
            pl.BlockSpec((C, 1), lambda b: (0, 0)),
        ],
        out_specs=pl.BlockSpec((tile, C, S), lambda b: (b, 0, 0)),
        compiler_params=pltpu.CompilerParams(
            dimension_semantics=("parallel",),
            vmem_limit_bytes=vmem_limit),
    )(xs, w1, b1.reshape(1, 1, Cr), w2.T, b2.reshape(C, 1))

    return out.reshape(B, C, H, W)
